# Initial kernel scaffold; baseline (speedup 1.0000x reference)
import jax
import jax.numpy as jnp
from jax.experimental import pallas as pl


def kernel(feat, edge_vals, gmask_rowsum, label_CSL, W_enc, W_dec, Wb1, b1, Wb2, b2, edge_index, gmask_index, assignment, perm_idx):
    raise NotImplementedError("write your pallas kernel here")


# speedup vs baseline: 1.8869x; 1.8869x over previous
import functools

import jax
import jax.numpy as jnp
from jax import lax
from jax.experimental import pallas as pl
from jax.experimental.pallas import tpu as pltpu
from jax.experimental.pallas import tpu_sc as plsc

N = 10000
IN_DIM = 256
LATENT = 512
E = 160000
M_SPOTS = 2000

NC, NS, LANES = 2, 16, 16
N_R = 10240
M_R = 2048
E_T = 10240
E_PAD = NS * E_T
NCH_E = 80
GRP = 8
NCH_N = 5
ROW_DUMP = N
SPOT_DUMP = M_SPOTS
RB = 1280
GRID_R = N_R // RB

_MESH = plsc.VectorSubcoreMesh(
    core_axis_name="c", subcore_axis_name="s", num_cores=NC, num_subcores=NS)

F32 = jnp.float32
I32 = jnp.int32


def _sds(shape, dtype=F32):
  return jax.ShapeDtypeStruct(shape, dtype)


def _fill_const(buf, value, rows, width):
  vec = jnp.full((LANES,), value, F32)

  def body(r, _):
    for k in range(width // LANES):
      buf[r, pl.ds(k * LANES, LANES)] = vec
    return 0

  lax.fori_loop(0, rows, body, 0)


def _zero_acc(acc, zbuf, sub, nacc_ch):
  for i in range(nacc_ch):
    pltpu.sync_copy(zbuf, acc.at[pl.ds((sub * nacc_ch + i) * 128, 128)])


def _scale_chunk(rows, vv, j):
  def q_body(q, _):
    val16 = vv[j, pl.ds(q * LANES, LANES)]
    for e in range(LANES):
      vb = jnp.broadcast_to(lax.slice(val16, (e,), (e + 1,)), (LANES,))
      r = q * LANES + e
      for k in range(128 // LANES):
        sl = (r, pl.ds(k * LANES, LANES))
        rows[sl] = rows[sl] * vb
    return 0

  lax.fori_loop(0, 8, q_body, 0)


def _segsum_task(tbl, outb, acc, gidx_t, sidx_t, vals_t,
                 gv, sv, vv, rows, sub, nchunks, nacc_ch, linear_gather):
  _fill_const(rows, 0.0, 128, 128)
  _zero_acc(acc, rows, sub, nacc_ch)
  plsc.subcore_barrier()
  lin_base = sub * nchunks * 128

  def chunk(base_j, jj, _):
    j = base_j + jj
    if linear_gather:
      pltpu.sync_copy(tbl.at[pl.ds(lin_base + j * 128, 128)], rows)
    else:
      pltpu.sync_copy(tbl.at[gv.at[jj]], rows)
    if vals_t is not None:
      _scale_chunk(rows, vv, jj)
    pltpu.sync_copy(rows, acc.at[sv.at[jj]], add=True)
    return 0

  def group(gi, _):
    off = gi * GRP
    if not linear_gather:
      pltpu.sync_copy(gidx_t.at[sub].at[pl.ds(off, GRP)], gv)
    pltpu.sync_copy(sidx_t.at[sub].at[pl.ds(off, GRP)], sv)
    if vals_t is not None:
      pltpu.sync_copy(vals_t.at[sub].at[pl.ds(off, GRP)], vv)
    lax.fori_loop(0, GRP, functools.partial(chunk, off), 0)
    return 0

  if nchunks % GRP == 0:
    lax.fori_loop(0, nchunks // GRP, group, 0)
  else:
    assert nchunks < GRP
    if not linear_gather:
      pltpu.sync_copy(gidx_t.at[sub], gv.at[pl.ds(0, nchunks)])
    pltpu.sync_copy(sidx_t.at[sub], sv.at[pl.ds(0, nchunks)])
    if vals_t is not None:
      pltpu.sync_copy(vals_t.at[sub], vv.at[pl.ds(0, nchunks)])
    lax.fori_loop(0, nchunks, functools.partial(chunk, 0), 0)
  plsc.subcore_barrier()
  for i in range(nacc_ch):
    off = (sub * nacc_ch + i) * 128
    pltpu.sync_copy(acc.at[pl.ds(off, 128)], outb.at[pl.ds(off, 128)])
  plsc.subcore_barrier()


def _gather_task(tbl, outb, gidx_t, gv, rows, sub, nchunks):
  pltpu.sync_copy(gidx_t.at[sub], gv.at[pl.ds(0, nchunks)])
  base = sub * nchunks * 128

  def chunk(j, _):
    pltpu.sync_copy(tbl.at[gv.at[j]], rows)
    pltpu.sync_copy(rows, outb.at[pl.ds(base + j * 128, 128)])
    return 0

  lax.fori_loop(0, nchunks, chunk, 0)


def _sc_perm_gather(z_blocks, perm_t):
  def body(*refs):
    zb = refs[0:4]
    permr = refs[4]
    za = refs[5:9]
    gv, rows = refs[9], refs[10]
    core = lax.axis_index("c")
    sub = lax.axis_index("s")
    for b in range(4):
      @pl.when(core == (b % 2))
      def _(b=b):
        _gather_task(zb[b], za[b], permr, gv, rows, sub, NCH_N)

  f = pl.kernel(
      body,
      out_type=[_sds((N_R, 128)) for _ in range(4)],
      mesh=_MESH,
      scratch_types=[pltpu.VMEM((NCH_N, 128), I32),
                     pltpu.VMEM((128, 128), F32)],
  )
  return f(*z_blocks, perm_t)


def _sc_spmm(z_blocks, za_blocks, adj_g, adj_s, adj_v):
  def body(*refs):
    zb = refs[0:4]
    zab = refs[4:8]
    gidx, sidx, vals = refs[8], refs[9], refs[10]
    h = refs[11:15]
    ha = refs[15:19]
    acc, gv, sv, vv, rows = refs[19:24]
    core = lax.axis_index("c")
    sub = lax.axis_index("s")
    for tb, ob in [(zb, h), (zab, ha)]:
      for b in range(4):
        @pl.when(core == (b % 2))
        def _(tb=tb, ob=ob, b=b):
          _segsum_task(tb[b], ob[b], acc, gidx, sidx, vals,
                       gv, sv, vv, rows, sub, NCH_E, NCH_N, False)

  f = pl.kernel(
      body,
      out_type=[_sds((N_R, 128)) for _ in range(8)],
      mesh=_MESH,
      scratch_types=[
          pltpu.VMEM_SHARED((N_R, 128), F32),
          pltpu.VMEM((GRP, 128), I32),
          pltpu.VMEM((GRP, 128), I32),
          pltpu.VMEM((GRP, 128), F32),
          pltpu.VMEM((128, 128), F32),
      ],
  )
  outs = f(*z_blocks, *za_blocks, adj_g, adj_s, adj_v)
  return outs[0:4], outs[4:8]


def _sc_phase2(d_blocks, h_blocks, ha_blocks, hr_blocks, har_blocks,
               adj_g, adj_s, adj_v, gm_g, gm_s, assign_t):
  def body(*refs):
    db = refs[0:2]
    hb = refs[2:6]
    hab = refs[6:10]
    hrb = refs[10:14]
    harb = refs[14:18]
    agidx, asidx, avals, ggidx, gsidx, asg = refs[18:24]
    outb = refs[24:26]
    G = refs[26:30]
    Ga = refs[30:34]
    S = refs[34:38]
    Sa = refs[38:42]
    cnt = refs[42]
    acc, cacc, gv, sv, vv, rows = refs[43:49]
    core = lax.axis_index("c")
    sub = lax.axis_index("s")
    for b in range(2):
      @pl.when(core == b)
      def _(b=b):
        _segsum_task(db[b], outb[b], acc, agidx, asidx, avals,
                     gv, sv, vv, rows, sub, NCH_E, NCH_N, False)
    for tb, ob in [(hrb, G), (harb, Ga)]:
      for b in range(4):
        @pl.when(core == (b % 2))
        def _(tb=tb, ob=ob, b=b):
          _segsum_task(tb[b], ob[b], acc, ggidx, gsidx, None,
                       gv, sv, vv, rows, sub, NCH_E, NCH_N, False)
    for tb, ob in [(hb, S), (hab, Sa)]:
      for b in range(4):
        @pl.when(core == (b % 2))
        def _(tb=tb, ob=ob, b=b):
          _segsum_task(tb[b], ob[b], acc, None, asg, None,
                       gv, sv, vv, rows, sub, NCH_N, 1, True)
    @pl.when(core == 0)
    def _():
      _fill_const(rows, 0.0, 128, 128)
      _zero_acc(cacc, rows, sub, 1)
      plsc.subcore_barrier()
      _fill_const(rows, 1.0, 128, 128)
      pltpu.sync_copy(asg.at[sub], sv.at[pl.ds(0, NCH_N)])

      def chunk(j, _):
        pltpu.sync_copy(rows, cacc.at[sv.at[j]], add=True)
        return 0

      lax.fori_loop(0, NCH_N, chunk, 0)
      plsc.subcore_barrier()
      pltpu.sync_copy(cacc.at[pl.ds(sub * 128, 128)],
                      cnt.at[pl.ds(sub * 128, 128)])

  f = pl.kernel(
      body,
      out_type=([_sds((N_R, 128)) for _ in range(2)]
                + [_sds((N_R, 128)) for _ in range(8)]
                + [_sds((M_R, 128)) for _ in range(8)]
                + [_sds((M_R, 128))]),
      mesh=_MESH,
      scratch_types=[
          pltpu.VMEM_SHARED((N_R, 128), F32),
          pltpu.VMEM_SHARED((M_R, 128), F32),
          pltpu.VMEM((GRP, 128), I32),
          pltpu.VMEM((GRP, 128), I32),
          pltpu.VMEM((GRP, 128), F32),
          pltpu.VMEM((128, 128), F32),
      ],
  )
  outs = f(*d_blocks, *h_blocks, *ha_blocks, *hr_blocks, *har_blocks,
           adj_g, adj_s, adj_v, gm_g, gm_s, assign_t)
  return outs[0:2], outs[2:6], outs[6:10], outs[10:14], outs[14:18], outs[18]


def _sc_spot_gather(s_blocks, sa_blocks, cnt, assign_t):
  def body(*refs):
    sb = refs[0:4]
    sab = refs[4:8]
    cnt_r = refs[8]
    asg = refs[9]
    sg = refs[10:14]
    sag = refs[14:18]
    cg = refs[18]
    gv, rows = refs[19:21]
    core = lax.axis_index("c")
    sub = lax.axis_index("s")
    for tb, ob in [(sb, sg), (sab, sag)]:
      for b in range(4):
        @pl.when(core == (b % 2))
        def _(tb=tb, ob=ob, b=b):
          _gather_task(tb[b], ob[b], asg, gv, rows, sub, NCH_N)

    @pl.when(core == 0)
    def _():
      pltpu.sync_copy(asg.at[sub], gv.at[pl.ds(0, NCH_N)])
      base = sub * NCH_N * 128

      def chunk(j, _):
        pltpu.sync_copy(cnt_r.at[gv.at[j]], rows)
        pltpu.sync_copy(rows, cg.at[pl.ds(base + j * 128, 128)])
        return 0

      lax.fori_loop(0, NCH_N, chunk, 0)

  f = pl.kernel(
      body,
      out_type=[_sds((N_R, 128)) for _ in range(8)] + [_sds((N_R, 128))],
      mesh=_MESH,
      scratch_types=[pltpu.VMEM((NCH_N, 128), I32),
                     pltpu.VMEM((128, 128), F32)],
  )
  outs = f(*s_blocks, *sa_blocks, cnt, assign_t)
  return outs[0:4], outs[4:8], outs[8]


def _tc_encode(feat_p, W_enc):
  def body(f_ref, w_ref, *z_refs):
    z = jnp.dot(f_ref[...], w_ref[...], preferred_element_type=F32)
    for cb in range(4):
      z_refs[cb][...] = z[:, cb * 128:(cb + 1) * 128]

  return pl.pallas_call(
      body,
      grid=(GRID_R,),
      in_specs=[pl.BlockSpec((RB, IN_DIM), lambda r: (r, 0)),
                pl.BlockSpec((IN_DIM, LATENT), lambda r: (0, 0))],
      out_specs=[pl.BlockSpec((RB, 128), lambda r: (r, 0)) for _ in range(4)],
      out_shape=[_sds((N_R, 128)) for _ in range(4)],
  )(feat_p, W_enc)


def _tc_phase2_mm(h_blocks, ha_blocks, W_dec, Wb1, Wb2):
  def body(*refs):
    hb = refs[0:4]
    hab = refs[4:8]
    wd, w1, w2 = refs[8], refs[9], refs[10]
    outs = refs[11:]
    d_r = outs[0:2]
    t_r = outs[2:6]
    ta_r = outs[6:10]
    u_r = outs[10:14]
    ua_r = outs[14:18]
    hr_r = outs[18:22]
    har_r = outs[22:26]
    x = jnp.concatenate([r[...] for r in hb], axis=1)
    xa = jnp.concatenate([r[...] for r in hab], axis=1)
    xr = jnp.maximum(x, 0.0)
    xar = jnp.maximum(xa, 0.0)
    d = jnp.dot(x, wd[...], preferred_element_type=F32)
    t = jnp.dot(xr, w1[...], preferred_element_type=F32)
    ta = jnp.dot(xar, w1[...], preferred_element_type=F32)
    u = jnp.dot(x, w2[...], preferred_element_type=F32)
    ua = jnp.dot(xa, w2[...], preferred_element_type=F32)
    for cb in range(2):
      d_r[cb][...] = d[:, cb * 128:(cb + 1) * 128]
    for cb in range(4):
      sl = slice(cb * 128, (cb + 1) * 128)
      t_r[cb][...] = t[:, sl]
      ta_r[cb][...] = ta[:, sl]
      u_r[cb][...] = u[:, sl]
      ua_r[cb][...] = ua[:, sl]
      hr_r[cb][...] = xr[:, sl]
      har_r[cb][...] = xar[:, sl]

  blk = lambda: pl.BlockSpec((RB, 128), lambda r: (r, 0))
  outs = pl.pallas_call(
      body,
      grid=(GRID_R,),
      in_specs=([blk() for _ in range(8)]
                + [pl.BlockSpec((LATENT, IN_DIM), lambda r: (0, 0)),
                   pl.BlockSpec((LATENT, LATENT), lambda r: (0, 0)),
                   pl.BlockSpec((LATENT, LATENT), lambda r: (0, 0))]),
      out_specs=[blk() for _ in range(26)],
      out_shape=[_sds((N_R, 128)) for _ in range(26)],
  )(*h_blocks, *ha_blocks, W_dec, Wb1, Wb2)
  return (outs[0:2], outs[2:6], outs[6:10], outs[10:14], outs[14:18],
          outs[18:22], outs[22:26])


def _tc_loss(feat_p, out_blocks, label_p, rowsum_p, G, Ga, T, Ta, U, Ua,
             SG, SAG, cg, b1, b2):
  def body(*refs):
    (f_ref, o0, o1, lab, rs) = refs[0:5]
    Gb = refs[5:9]
    Gab = refs[9:13]
    Tb = refs[13:17]
    Tab = refs[17:21]
    Ub = refs[21:25]
    Uab = refs[25:29]
    SGb = refs[29:33]
    SAGb = refs[33:37]
    cg_r, b1_r, b2_r = refs[37], refs[38], refs[39]
    out_ref = refs[40]
    r = pl.program_id(0)

    cat = lambda bs: jnp.concatenate([x[...] for x in bs], axis=1)
    rows = jax.lax.broadcasted_iota(I32, (RB, 1), 0) + r * RB
    mask = (rows < N)[:, 0]

    rsv = rs[...]
    x = cat(Gb) / rsv
    nrm = jnp.sqrt(jnp.sum(x * x, axis=1, keepdims=True))
    g = jax.nn.sigmoid(x / jnp.maximum(nrm, 1e-12))
    xa = cat(Gab) / rsv
    nrma = jnp.sqrt(jnp.sum(xa * xa, axis=1, keepdims=True))
    ga = jax.nn.sigmoid(xa / jnp.maximum(nrma, 1e-12))

    cgv = cg_r[:, 0:1] + 1e-08
    c = jax.nn.sigmoid(cat(SGb) / cgv)
    ca = jax.nn.sigmoid(cat(SAGb) / cgv)

    t = cat(Tb)
    ta = cat(Tab)
    u = cat(Ub)
    ua = cat(Uab)
    b1v = b1_r[0, 0]
    b2v = b2_r[0, 0]

    def bce_sum(x, y):
      v = jnp.maximum(x, 0.0) - x * y + jnp.log1p(jnp.exp(-jnp.abs(x)))
      return jnp.sum(jnp.where(mask, v, 0.0))

    y0 = lab[:, 0]
    y1 = lab[:, 1]
    sl = (bce_sum(jnp.sum(t * g, axis=1) + b1v, y0)
          + bce_sum(jnp.sum(ta * g, axis=1) + b1v, y1)
          + bce_sum(jnp.sum(ta * ga, axis=1) + b1v, y0)
          + bce_sum(jnp.sum(t * ga, axis=1) + b1v, y1))
    sh = (bce_sum(jnp.sum(u * c, axis=1) + b2v, y0)
          + bce_sum(jnp.sum(ua * c, axis=1) + b2v, y1)
          + bce_sum(jnp.sum(ua * ca, axis=1) + b2v, y0)
          + bce_sum(jnp.sum(u * ca, axis=1) + b2v, y1))
    diff = f_ref[...] - jnp.concatenate([o0[...], o1[...]], axis=1)
    mse = jnp.sum(jnp.where(mask, jnp.sum(diff * diff, axis=1), 0.0))
    total = (sl + sh) / (2.0 * N) + mse / (N * IN_DIM)

    total2d = total[None, None]

    @pl.when(r == 0)
    def _():
      out_ref[...] = total2d

    @pl.when(r > 0)
    def _():
      out_ref[...] = out_ref[...] + total2d

  blk = lambda: pl.BlockSpec((RB, 128), lambda r: (r, 0))
  return pl.pallas_call(
      body,
      grid=(GRID_R,),
      in_specs=([pl.BlockSpec((RB, IN_DIM), lambda r: (r, 0)),
                 blk(), blk(),
                 pl.BlockSpec((RB, 2), lambda r: (r, 0)),
                 pl.BlockSpec((RB, 1), lambda r: (r, 0))]
                + [blk() for _ in range(32)]
                + [pl.BlockSpec((RB, 128), lambda r: (r, 0)),
                   pl.BlockSpec((1, 1), lambda r: (0, 0)),
                   pl.BlockSpec((1, 1), lambda r: (0, 0))]),
      out_specs=pl.BlockSpec((1, 1), lambda r: (0, 0)),
      out_shape=_sds((1, 1)),
  )(feat_p, *out_blocks, label_p, rowsum_p, *G, *Ga, *T, *Ta, *U, *Ua,
    *SG, *SAG, cg, b1, b2)


def kernel(feat, edge_vals, gmask_rowsum, label_CSL, W_enc, W_dec, Wb1, b1,
           Wb2, b2, edge_index, gmask_index, assignment, perm_idx):
  feat_p = jnp.pad(feat, ((0, N_R - N), (0, 0)))
  label_p = jnp.pad(label_CSL, ((0, N_R - N), (0, 0)))
  rowsum_p = jnp.pad(gmask_rowsum, ((0, N_R - N), (0, 0)), constant_values=1.0)

  def edge_tiles(idx2, vals=None):
    g = jnp.pad(idx2[1], (0, E_PAD - E)).reshape(NS, NCH_E, 128)
    s = jnp.pad(idx2[0], (0, E_PAD - E),
                constant_values=ROW_DUMP).reshape(NS, NCH_E, 128)
    if vals is None:
      return g, s
    v = jnp.pad(vals, (0, E_PAD - E)).reshape(NS, NCH_E, 128)
    return g, s, v

  adj_g, adj_s, adj_v = edge_tiles(edge_index, edge_vals)
  gm_g, gm_s = edge_tiles(gmask_index)
  assign_t = jnp.pad(assignment, (0, N_R - N),
                     constant_values=SPOT_DUMP).reshape(NS, NCH_N, 128)
  perm_t = jnp.pad(perm_idx, (0, N_R - N)).reshape(NS, NCH_N, 128)
  b1r = b1.reshape(1, 1)
  b2r = b2.reshape(1, 1)

  z = _tc_encode(feat_p, W_enc)
  za = _sc_perm_gather(z, perm_t)
  h, ha = _sc_spmm(z, za, adj_g, adj_s, adj_v)

  d, T, Ta, U, Ua, hr, har = _tc_phase2_mm(h, ha, W_dec, Wb1, Wb2)

  out_b, G, Ga, S, Sa, cnt = _sc_phase2(
      d, h, ha, hr, har, adj_g, adj_s, adj_v, gm_g, gm_s, assign_t)
  SG, SAG, cg = _sc_spot_gather(S, Sa, cnt, assign_t)

  res = _tc_loss(feat_p, out_b, label_p, rowsum_p, G, Ga, T, Ta, U, Ua,
                 SG, SAG, cg, b1r, b2r)
  return res[0, 0]


# speedup vs baseline: 2.6712x; 1.4157x over previous
import functools

import jax
import jax.numpy as jnp
from jax import lax
from jax.experimental import pallas as pl
from jax.experimental.pallas import tpu as pltpu
from jax.experimental.pallas import tpu_sc as plsc

N = 10000
IN_DIM = 256
LATENT = 512
E = 160000
M_SPOTS = 2000

NC, NS, LANES = 2, 16, 16
N_R = 10240
M_R = 2048
E_T = 10240
E_PAD = NS * E_T
NCH_E = 80
GRP = 16
NCH_N = 5
ROW_DUMP = N
SPOT_DUMP = M_SPOTS
RB = 1280
GRID_R = N_R // RB

_MESH = plsc.VectorSubcoreMesh(
    core_axis_name="c", subcore_axis_name="s", num_cores=NC, num_subcores=NS)

F32 = jnp.float32
I32 = jnp.int32


def _sds(shape, dtype=F32):
  return jax.ShapeDtypeStruct(shape, dtype)


def _fill_const(buf, value, rows, width):
  vec = jnp.full((LANES,), value, F32)

  def body(r, _):
    for k in range(width // LANES):
      buf[r, pl.ds(k * LANES, LANES)] = vec
    return 0

  lax.fori_loop(0, rows, body, 0)


def _zero_acc(acc, zbuf, sub, nacc_ch):
  for i in range(nacc_ch):
    pltpu.sync_copy(zbuf, acc.at[pl.ds((sub * nacc_ch + i) * 128, 128)])


def _scale_chunk(rows, vv, j):
  def q_body(q, _):
    val16 = vv[j, pl.ds(q * LANES, LANES)]
    for e in range(LANES):
      vb = jnp.broadcast_to(lax.slice(val16, (e,), (e + 1,)), (LANES,))
      r = q * LANES + e
      for k in range(128 // LANES):
        sl = (r, pl.ds(k * LANES, LANES))
        rows[sl] = rows[sl] * vb
    return 0

  lax.fori_loop(0, 8, q_body, 0)


def _edge_pipeline(tbl, acc, gidx_t, sidx_t, vals_t, gv, sv, vv,
                   rowsA, rowsB, gsA, gsB, ssA, ssB, sub):
  weighted = vals_t is not None

  def gather(jj, buf, sem):
    return pltpu.async_copy(tbl.at[gv.at[jj]], buf, sem)

  def wait_gather(jj, buf, sem):
    pltpu.make_async_copy(tbl.at[gv.at[jj]], buf, sem).wait()

  def scatter(jj, buf, sem):
    return pltpu.async_copy(buf, acc.at[sv.at[jj]], sem, add=True)

  def wait_scatter(jj, buf, sem):
    pltpu.make_async_copy(buf, acc.at[sv.at[jj]], sem).wait()

  def scale(jj, buf):
    if weighted:
      _scale_chunk(buf, vv, jj)

  def group(gi, _):
    off = gi * GRP
    pltpu.sync_copy(gidx_t.at[sub].at[pl.ds(off, GRP)], gv)
    pltpu.sync_copy(sidx_t.at[sub].at[pl.ds(off, GRP)], sv)
    if weighted:
      pltpu.sync_copy(vals_t.at[sub].at[pl.ds(off, GRP)], vv)
    gather(0, rowsA, gsA)
    gather(1, rowsB, gsB)
    wait_gather(0, rowsA, gsA)
    scale(0, rowsA)
    scatter(0, rowsA, ssA)

    def pair(pi, _):
      ja = 2 * pi
      jb = 2 * pi + 1
      wait_scatter(ja - 2, rowsA, ssA)
      gather(ja, rowsA, gsA)
      wait_gather(jb - 2, rowsB, gsB)
      scale(jb - 2, rowsB)
      scatter(jb - 2, rowsB, ssB)
      wait_gather(ja, rowsA, gsA)
      wait_scatter(jb - 2, rowsB, ssB)
      gather(jb, rowsB, gsB)
      scale(ja, rowsA)
      scatter(ja, rowsA, ssA)
      return 0

    lax.fori_loop(1, GRP // 2, pair, 0)
    wait_gather(GRP - 1, rowsB, gsB)
    scale(GRP - 1, rowsB)
    wait_scatter(GRP - 2, rowsA, ssA)
    scatter(GRP - 1, rowsB, ssB)
    wait_scatter(GRP - 1, rowsB, ssB)
    return 0

  lax.fori_loop(0, NCH_E // GRP, group, 0)


def _segsum_task(tbl, outb, acc, gidx_t, sidx_t, vals_t,
                 gv, sv, vv, rowsA, rowsB, gsA, gsB, ssA, ssB,
                 sub, nchunks, nacc_ch, linear_gather):
  _fill_const(rowsA, 0.0, 128, 128)
  _zero_acc(acc, rowsA, sub, nacc_ch)
  plsc.subcore_barrier()
  if nchunks == NCH_E and not linear_gather:
    _edge_pipeline(tbl, acc, gidx_t, sidx_t, vals_t, gv, sv, vv,
                   rowsA, rowsB, gsA, gsB, ssA, ssB, sub)
  else:
    assert nchunks < GRP
    lin_base = sub * nchunks * 128
    if not linear_gather:
      pltpu.sync_copy(gidx_t.at[sub], gv.at[pl.ds(0, nchunks)])
    pltpu.sync_copy(sidx_t.at[sub], sv.at[pl.ds(0, nchunks)])
    if vals_t is not None:
      pltpu.sync_copy(vals_t.at[sub], vv.at[pl.ds(0, nchunks)])

    def chunk(j, _):
      if linear_gather:
        pltpu.sync_copy(tbl.at[pl.ds(lin_base + j * 128, 128)], rowsA)
      else:
        pltpu.sync_copy(tbl.at[gv.at[j]], rowsA)
      if vals_t is not None:
        _scale_chunk(rowsA, vv, j)
      pltpu.sync_copy(rowsA, acc.at[sv.at[j]], add=True)
      return 0

    lax.fori_loop(0, nchunks, chunk, 0)
  plsc.subcore_barrier()
  for i in range(nacc_ch):
    off = (sub * nacc_ch + i) * 128
    pltpu.sync_copy(acc.at[pl.ds(off, 128)], outb.at[pl.ds(off, 128)])
  plsc.subcore_barrier()


def _gather_task(tbl, outb, gidx_t, gv, rows, sub, nchunks):
  pltpu.sync_copy(gidx_t.at[sub], gv.at[pl.ds(0, nchunks)])
  base = sub * nchunks * 128

  def chunk(j, _):
    pltpu.sync_copy(tbl.at[gv.at[j]], rows)
    pltpu.sync_copy(rows, outb.at[pl.ds(base + j * 128, 128)])
    return 0

  lax.fori_loop(0, nchunks, chunk, 0)


def _sc_perm_gather(z_blocks, perm_t):
  def body(*refs):
    zb = refs[0:4]
    permr = refs[4]
    za = refs[5:9]
    gv, rows = refs[9], refs[10]
    core = lax.axis_index("c")
    sub = lax.axis_index("s")
    for b in range(4):
      @pl.when(core == (b % 2))
      def _(b=b):
        _gather_task(zb[b], za[b], permr, gv, rows, sub, NCH_N)

  f = pl.kernel(
      body,
      out_type=[_sds((N_R, 128)) for _ in range(4)],
      mesh=_MESH,
      scratch_types=[pltpu.VMEM((NCH_N, 128), I32),
                     pltpu.VMEM((128, 128), F32)],
  )
  return f(*z_blocks, perm_t)


def _sc_spmm(z_blocks, za_blocks, adj_g, adj_s, adj_v):
  def body(*refs):
    zb = refs[0:4]
    zab = refs[4:8]
    gidx, sidx, vals = refs[8], refs[9], refs[10]
    h = refs[11:15]
    ha = refs[15:19]
    acc, gv, sv, vv, rowsA, rowsB, gsA, gsB, ssA, ssB = refs[19:29]
    core = lax.axis_index("c")
    sub = lax.axis_index("s")
    for tb, ob in [(zb, h), (zab, ha)]:
      for b in range(4):
        @pl.when(core == (b % 2))
        def _(tb=tb, ob=ob, b=b):
          _segsum_task(tb[b], ob[b], acc, gidx, sidx, vals,
                       gv, sv, vv, rowsA, rowsB, gsA, gsB, ssA, ssB,
                       sub, NCH_E, NCH_N, False)

  f = pl.kernel(
      body,
      out_type=[_sds((N_R, 128)) for _ in range(8)],
      mesh=_MESH,
      scratch_types=[
          pltpu.VMEM_SHARED((N_R, 128), F32),
          pltpu.VMEM((GRP, 128), I32),
          pltpu.VMEM((GRP, 128), I32),
          pltpu.VMEM((GRP, 128), F32),
          pltpu.VMEM((128, 128), F32),
          pltpu.VMEM((128, 128), F32),
          pltpu.SemaphoreType.DMA,
          pltpu.SemaphoreType.DMA,
          pltpu.SemaphoreType.DMA,
          pltpu.SemaphoreType.DMA,
      ],
  )
  outs = f(*z_blocks, *za_blocks, adj_g, adj_s, adj_v)
  return outs[0:4], outs[4:8]


def _sc_phase2(d_blocks, h_blocks, ha_blocks, hr_blocks, har_blocks,
               adj_g, adj_s, adj_v, gm_g, gm_s, assign_t):
  def body(*refs):
    db = refs[0:2]
    hb = refs[2:6]
    hab = refs[6:10]
    hrb = refs[10:14]
    harb = refs[14:18]
    agidx, asidx, avals, ggidx, gsidx, asg = refs[18:24]
    outb = refs[24:26]
    G = refs[26:30]
    Ga = refs[30:34]
    S = refs[34:38]
    Sa = refs[38:42]
    cnt = refs[42]
    acc, gv, sv, vv, rowsA, rowsB, gsA, gsB, ssA, ssB = refs[43:53]
    core = lax.axis_index("c")
    sub = lax.axis_index("s")
    for b in range(2):
      @pl.when(core == b)
      def _(b=b):
        _segsum_task(db[b], outb[b], acc, agidx, asidx, avals,
                     gv, sv, vv, rowsA, rowsB, gsA, gsB, ssA, ssB,
                     sub, NCH_E, NCH_N, False)
    for tb, ob in [(hrb, G), (harb, Ga)]:
      for b in range(4):
        @pl.when(core == (b % 2))
        def _(tb=tb, ob=ob, b=b):
          _segsum_task(tb[b], ob[b], acc, ggidx, gsidx, None,
                       gv, sv, vv, rowsA, rowsB, gsA, gsB, ssA, ssB,
                       sub, NCH_E, NCH_N, False)
    for tb, ob in [(hb, S), (hab, Sa)]:
      for b in range(4):
        @pl.when(core == (b % 2))
        def _(tb=tb, ob=ob, b=b):
          _segsum_task(tb[b], ob[b], acc, None, asg, None,
                       gv, sv, vv, rowsA, rowsB, gsA, gsB, ssA, ssB,
                       sub, NCH_N, 1, True)
    @pl.when(core == 0)
    def _():
      _fill_const(rowsA, 0.0, 128, 128)
      _zero_acc(acc, rowsA, sub, 1)
      plsc.subcore_barrier()
      _fill_const(rowsA, 1.0, 128, 128)
      pltpu.sync_copy(asg.at[sub], sv.at[pl.ds(0, NCH_N)])

      def chunk(j, _):
        pltpu.sync_copy(rowsA, acc.at[sv.at[j]], add=True)
        return 0

      lax.fori_loop(0, NCH_N, chunk, 0)
      plsc.subcore_barrier()
      pltpu.sync_copy(acc.at[pl.ds(sub * 128, 128)],
                      cnt.at[pl.ds(sub * 128, 128)])

  f = pl.kernel(
      body,
      out_type=([_sds((N_R, 128)) for _ in range(2)]
                + [_sds((N_R, 128)) for _ in range(8)]
                + [_sds((M_R, 128)) for _ in range(8)]
                + [_sds((M_R, 128))]),
      mesh=_MESH,
      scratch_types=[
          pltpu.VMEM_SHARED((N_R, 128), F32),
          pltpu.VMEM((GRP, 128), I32),
          pltpu.VMEM((GRP, 128), I32),
          pltpu.VMEM((GRP, 128), F32),
          pltpu.VMEM((128, 128), F32),
          pltpu.VMEM((128, 128), F32),
          pltpu.SemaphoreType.DMA,
          pltpu.SemaphoreType.DMA,
          pltpu.SemaphoreType.DMA,
          pltpu.SemaphoreType.DMA,
      ],
  )
  outs = f(*d_blocks, *h_blocks, *ha_blocks, *hr_blocks, *har_blocks,
           adj_g, adj_s, adj_v, gm_g, gm_s, assign_t)
  return outs[0:2], outs[2:6], outs[6:10], outs[10:14], outs[14:18], outs[18]


def _sc_spot_gather(s_blocks, sa_blocks, cnt, assign_t):
  def body(*refs):
    sb = refs[0:4]
    sab = refs[4:8]
    cnt_r = refs[8]
    asg = refs[9]
    sg = refs[10:14]
    sag = refs[14:18]
    cg = refs[18]
    gv, rows = refs[19:21]
    core = lax.axis_index("c")
    sub = lax.axis_index("s")
    for tb, ob in [(sb, sg), (sab, sag)]:
      for b in range(4):
        @pl.when(core == (b % 2))
        def _(tb=tb, ob=ob, b=b):
          _gather_task(tb[b], ob[b], asg, gv, rows, sub, NCH_N)

    @pl.when(core == 0)
    def _():
      pltpu.sync_copy(asg.at[sub], gv.at[pl.ds(0, NCH_N)])
      base = sub * NCH_N * 128

      def chunk(j, _):
        pltpu.sync_copy(cnt_r.at[gv.at[j]], rows)
        pltpu.sync_copy(rows, cg.at[pl.ds(base + j * 128, 128)])
        return 0

      lax.fori_loop(0, NCH_N, chunk, 0)

  f = pl.kernel(
      body,
      out_type=[_sds((N_R, 128)) for _ in range(8)] + [_sds((N_R, 128))],
      mesh=_MESH,
      scratch_types=[pltpu.VMEM((NCH_N, 128), I32),
                     pltpu.VMEM((128, 128), F32)],
  )
  outs = f(*s_blocks, *sa_blocks, cnt, assign_t)
  return outs[0:4], outs[4:8], outs[8]


def _tc_encode(feat_p, W_enc):
  def body(f_ref, w_ref, *z_refs):
    z = jnp.dot(f_ref[...], w_ref[...], preferred_element_type=F32)
    for cb in range(4):
      z_refs[cb][...] = z[:, cb * 128:(cb + 1) * 128]

  return pl.pallas_call(
      body,
      grid=(GRID_R,),
      in_specs=[pl.BlockSpec((RB, IN_DIM), lambda r: (r, 0)),
                pl.BlockSpec((IN_DIM, LATENT), lambda r: (0, 0))],
      out_specs=[pl.BlockSpec((RB, 128), lambda r: (r, 0)) for _ in range(4)],
      out_shape=[_sds((N_R, 128)) for _ in range(4)],
  )(feat_p, W_enc)


def _tc_phase2_mm(h_blocks, ha_blocks, W_dec, Wb1, Wb2):
  def body(*refs):
    hb = refs[0:4]
    hab = refs[4:8]
    wd, w1, w2 = refs[8], refs[9], refs[10]
    outs = refs[11:]
    d_r = outs[0:2]
    t_r = outs[2:6]
    ta_r = outs[6:10]
    u_r = outs[10:14]
    ua_r = outs[14:18]
    hr_r = outs[18:22]
    har_r = outs[22:26]
    x = jnp.concatenate([r[...] for r in hb], axis=1)
    xa = jnp.concatenate([r[...] for r in hab], axis=1)
    xr = jnp.maximum(x, 0.0)
    xar = jnp.maximum(xa, 0.0)
    d = jnp.dot(x, wd[...], preferred_element_type=F32)
    t = jnp.dot(xr, w1[...], preferred_element_type=F32)
    ta = jnp.dot(xar, w1[...], preferred_element_type=F32)
    u = jnp.dot(x, w2[...], preferred_element_type=F32)
    ua = jnp.dot(xa, w2[...], preferred_element_type=F32)
    for cb in range(2):
      d_r[cb][...] = d[:, cb * 128:(cb + 1) * 128]
    for cb in range(4):
      sl = slice(cb * 128, (cb + 1) * 128)
      t_r[cb][...] = t[:, sl]
      ta_r[cb][...] = ta[:, sl]
      u_r[cb][...] = u[:, sl]
      ua_r[cb][...] = ua[:, sl]
      hr_r[cb][...] = xr[:, sl]
      har_r[cb][...] = xar[:, sl]

  blk = lambda: pl.BlockSpec((RB, 128), lambda r: (r, 0))
  outs = pl.pallas_call(
      body,
      grid=(GRID_R,),
      in_specs=([blk() for _ in range(8)]
                + [pl.BlockSpec((LATENT, IN_DIM), lambda r: (0, 0)),
                   pl.BlockSpec((LATENT, LATENT), lambda r: (0, 0)),
                   pl.BlockSpec((LATENT, LATENT), lambda r: (0, 0))]),
      out_specs=[blk() for _ in range(26)],
      out_shape=[_sds((N_R, 128)) for _ in range(26)],
  )(*h_blocks, *ha_blocks, W_dec, Wb1, Wb2)
  return (outs[0:2], outs[2:6], outs[6:10], outs[10:14], outs[14:18],
          outs[18:22], outs[22:26])


def _tc_loss(feat_p, out_blocks, label_p, rowsum_p, G, Ga, T, Ta, U, Ua,
             SG, SAG, cg, b1, b2):
  def body(*refs):
    (f_ref, o0, o1, lab, rs) = refs[0:5]
    Gb = refs[5:9]
    Gab = refs[9:13]
    Tb = refs[13:17]
    Tab = refs[17:21]
    Ub = refs[21:25]
    Uab = refs[25:29]
    SGb = refs[29:33]
    SAGb = refs[33:37]
    cg_r, b1_r, b2_r = refs[37], refs[38], refs[39]
    out_ref = refs[40]
    r = pl.program_id(0)

    cat = lambda bs: jnp.concatenate([x[...] for x in bs], axis=1)
    rows = jax.lax.broadcasted_iota(I32, (RB, 1), 0) + r * RB
    mask = (rows < N)[:, 0]

    rsv = rs[...]
    x = cat(Gb) / rsv
    nrm = jnp.sqrt(jnp.sum(x * x, axis=1, keepdims=True))
    g = jax.nn.sigmoid(x / jnp.maximum(nrm, 1e-12))
    xa = cat(Gab) / rsv
    nrma = jnp.sqrt(jnp.sum(xa * xa, axis=1, keepdims=True))
    ga = jax.nn.sigmoid(xa / jnp.maximum(nrma, 1e-12))

    cgv = cg_r[:, 0:1] + 1e-08
    c = jax.nn.sigmoid(cat(SGb) / cgv)
    ca = jax.nn.sigmoid(cat(SAGb) / cgv)

    t = cat(Tb)
    ta = cat(Tab)
    u = cat(Ub)
    ua = cat(Uab)
    b1v = b1_r[0, 0]
    b2v = b2_r[0, 0]

    def bce_sum(x, y):
      v = jnp.maximum(x, 0.0) - x * y + jnp.log1p(jnp.exp(-jnp.abs(x)))
      return jnp.sum(jnp.where(mask, v, 0.0))

    y0 = lab[:, 0]
    y1 = lab[:, 1]
    sl = (bce_sum(jnp.sum(t * g, axis=1) + b1v, y0)
          + bce_sum(jnp.sum(ta * g, axis=1) + b1v, y1)
          + bce_sum(jnp.sum(ta * ga, axis=1) + b1v, y0)
          + bce_sum(jnp.sum(t * ga, axis=1) + b1v, y1))
    sh = (bce_sum(jnp.sum(u * c, axis=1) + b2v, y0)
          + bce_sum(jnp.sum(ua * c, axis=1) + b2v, y1)
          + bce_sum(jnp.sum(ua * ca, axis=1) + b2v, y0)
          + bce_sum(jnp.sum(u * ca, axis=1) + b2v, y1))
    diff = f_ref[...] - jnp.concatenate([o0[...], o1[...]], axis=1)
    mse = jnp.sum(jnp.where(mask, jnp.sum(diff * diff, axis=1), 0.0))
    total = (sl + sh) / (2.0 * N) + mse / (N * IN_DIM)

    total2d = total[None, None]

    @pl.when(r == 0)
    def _():
      out_ref[...] = total2d

    @pl.when(r > 0)
    def _():
      out_ref[...] = out_ref[...] + total2d

  blk = lambda: pl.BlockSpec((RB, 128), lambda r: (r, 0))
  return pl.pallas_call(
      body,
      grid=(GRID_R,),
      in_specs=([pl.BlockSpec((RB, IN_DIM), lambda r: (r, 0)),
                 blk(), blk(),
                 pl.BlockSpec((RB, 2), lambda r: (r, 0)),
                 pl.BlockSpec((RB, 1), lambda r: (r, 0))]
                + [blk() for _ in range(32)]
                + [pl.BlockSpec((RB, 128), lambda r: (r, 0)),
                   pl.BlockSpec((1, 1), lambda r: (0, 0)),
                   pl.BlockSpec((1, 1), lambda r: (0, 0))]),
      out_specs=pl.BlockSpec((1, 1), lambda r: (0, 0)),
      out_shape=_sds((1, 1)),
  )(feat_p, *out_blocks, label_p, rowsum_p, *G, *Ga, *T, *Ta, *U, *Ua,
    *SG, *SAG, cg, b1, b2)


def kernel(feat, edge_vals, gmask_rowsum, label_CSL, W_enc, W_dec, Wb1, b1,
           Wb2, b2, edge_index, gmask_index, assignment, perm_idx):
  feat_p = jnp.pad(feat, ((0, N_R - N), (0, 0)))
  label_p = jnp.pad(label_CSL, ((0, N_R - N), (0, 0)))
  rowsum_p = jnp.pad(gmask_rowsum, ((0, N_R - N), (0, 0)), constant_values=1.0)

  e_real = E // NS
  e_pad = E_T - e_real
  epad_s = jnp.broadcast_to(
      ROW_DUMP + (jnp.arange(e_pad, dtype=I32) % (N_R - N))[None, :],
      (NS, e_pad))
  zpad_i = jnp.zeros((NS, e_pad), I32)
  zpad_f = jnp.zeros((NS, e_pad), F32)

  def edge_tiles(idx2, vals=None):
    g = jnp.concatenate([idx2[1].reshape(NS, e_real), zpad_i],
                        axis=1).reshape(NS, NCH_E, 128)
    srt = jnp.concatenate([idx2[0].reshape(NS, e_real), epad_s],
                          axis=1).reshape(NS, NCH_E, 128)
    if vals is None:
      return g, srt
    v = jnp.concatenate([vals.reshape(NS, e_real), zpad_f],
                        axis=1).reshape(NS, NCH_E, 128)
    return g, srt, v

  adj_g, adj_s, adj_v = edge_tiles(edge_index, edge_vals)
  gm_g, gm_s = edge_tiles(gmask_index)
  spot_dump = SPOT_DUMP + (jnp.arange(N_R - N, dtype=I32) % (M_R - M_SPOTS))
  assign_t = jnp.concatenate(
      [assignment, spot_dump]).reshape(NS, NCH_N, 128)
  perm_t = jnp.pad(perm_idx, (0, N_R - N)).reshape(NS, NCH_N, 128)
  b1r = b1.reshape(1, 1)
  b2r = b2.reshape(1, 1)

  z = _tc_encode(feat_p, W_enc)
  za = _sc_perm_gather(z, perm_t)
  h, ha = _sc_spmm(z, za, adj_g, adj_s, adj_v)

  d, T, Ta, U, Ua, hr, har = _tc_phase2_mm(h, ha, W_dec, Wb1, Wb2)

  out_b, G, Ga, S, Sa, cnt = _sc_phase2(
      d, h, ha, hr, har, adj_g, adj_s, adj_v, gm_g, gm_s, assign_t)
  SG, SAG, cg = _sc_spot_gather(S, Sa, cnt, assign_t)

  res = _tc_loss(feat_p, out_b, label_p, rowsum_p, G, Ga, T, Ta, U, Ua,
                 SG, SAG, cg, b1r, b2r)
  return res[0, 0]
